# baseline (speedup 1.0000x reference)
import functools

import jax
import jax.numpy as jnp
from jax import lax
from jax.experimental import pallas as pl
from jax.experimental.pallas import tpu as pltpu
from jax.experimental.pallas import tpu_sc as plsc

N = 10000
E = 320000
NC = 2
NS = 16
NW = NC * NS
N_PAD = 10240
ROWS_PER_TILE = N_PAD // NS
EPW = E // NW
KD = 80
DRSTEPS = EPW // KD
DSTEPS = 128
DPSTEPS = DSTEPS - DRSTEPS
K = 128
STEPS = 80
EPAD = STEPS * K - EPW
NB = 1
GROUPS = STEPS // NB
PAIRS = GROUPS // 2
RING = 4

_mesh = plsc.VectorSubcoreMesh(core_axis_name="c", subcore_axis_name="s")


@functools.partial(
    pl.kernel,
    out_type=jax.ShapeDtypeStruct((NC * N_PAD,), jnp.float32),
    mesh=_mesh,
    scratch_types=[
        pltpu.VMEM((DSTEPS, KD), jnp.int32),
        pltpu.VMEM((KD,), jnp.float32),
        pltpu.VMEM((ROWS_PER_TILE,), jnp.float32),
        pltpu.VMEM_SHARED((N_PAD,), jnp.float32),
        pltpu.SemaphoreType.DMA,
    ],
)
def _deg_kernel(dst_hbm, dpad_hbm, out_hbm, didx_v, ones_v, zero_v, hist_sh,
                sem):
    c = lax.axis_index("c")
    s = lax.axis_index("s")
    wid = s * NC + c
    for i in range(ROWS_PER_TILE // 16):
        zero_v[pl.ds(16 * i, 16)] = jnp.zeros((16,), jnp.float32)
    for i in range(KD // 16):
        ones_v[pl.ds(16 * i, 16)] = jnp.ones((16,), jnp.float32)
    pltpu.sync_copy(dst_hbm.at[wid], didx_v.at[pl.ds(0, DRSTEPS)])
    pltpu.sync_copy(dpad_hbm.at[wid], didx_v.at[pl.ds(DRSTEPS, DPSTEPS)])
    pltpu.sync_copy(zero_v, hist_sh.at[pl.ds(ROWS_PER_TILE * s, ROWS_PER_TILE)])
    plsc.subcore_barrier()

    def fire(i, carry):
        pltpu.async_copy(ones_v, hist_sh.at[didx_v.at[i]], sem, add=True)
        return carry

    lax.fori_loop(0, DSTEPS, fire, 0)

    def drain(i, carry):
        pltpu.make_async_copy(ones_v, hist_sh.at[didx_v.at[i]], sem).wait()
        return carry

    lax.fori_loop(0, DSTEPS, drain, 0)
    plsc.subcore_barrier()
    pltpu.sync_copy(
        hist_sh.at[pl.ds(ROWS_PER_TILE * s, ROWS_PER_TILE)],
        out_hbm.at[pl.ds(c * N_PAD + ROWS_PER_TILE * s, ROWS_PER_TILE)],
    )


@functools.partial(
    pl.kernel,
    out_type=jax.ShapeDtypeStruct((NC * N_PAD, 128), jnp.float32),
    mesh=_mesh,
    scratch_types=[
        pltpu.VMEM((STEPS * K,), jnp.int32),
        pltpu.VMEM((RING * NB, K), jnp.int32),
        pltpu.VMEM((2, NB, K, 128), jnp.float32),
        pltpu.VMEM_SHARED((N_PAD, 128), jnp.float32),
        pltpu.SemaphoreType.DMA,
        pltpu.SemaphoreType.DMA,
        pltpu.SemaphoreType.DMA,
        pltpu.SemaphoreType.DMA,
    ],
)
def _agg_kernel(g_hbm, srcf_hbm, dst3_hbm, z_hbm, out_hbm,
                sidx_v, dring_v, rows_v, acc_sh, gsem_a, gsem_b, ssem, isem):
    c = lax.axis_index("c")
    s = lax.axis_index("s")
    wid = s * NC + c
    r0 = ROWS_PER_TILE * s
    gsems = (gsem_a, gsem_b)

    pltpu.sync_copy(srcf_hbm.at[wid], sidx_v)
    pltpu.sync_copy(dst3_hbm.at[wid, pl.ds(0, RING * NB)], dring_v)
    pltpu.sync_copy(z_hbm.at[pl.ds(r0, ROWS_PER_TILE)],
                    acc_sh.at[pl.ds(r0, ROWS_PER_TILE)])
    plsc.subcore_barrier()

    for st in range(2):
        for b in range(NB):
            i = st * NB + b
            pltpu.async_copy(g_hbm.at[sidx_v.at[pl.ds(i * K, K)]],
                             rows_v.at[st, b], gsems[st])

    def pair(m, carry):
        for st in range(2):
            g = 2 * m + st
            slot = lax.rem(g, RING) * NB

            @pl.when(g >= RING)
            def _():
                pltpu.make_async_copy(
                    dst3_hbm.at[wid, pl.ds(g * NB, NB)],
                    dring_v.at[pl.ds(slot, NB)], isem).wait()
            for b in range(NB):
                i = g * NB + b
                pltpu.make_async_copy(g_hbm.at[sidx_v.at[pl.ds(i * K, K)]],
                                      rows_v.at[st, b], gsems[st]).wait()
                pltpu.async_copy(rows_v.at[st, b],
                                 acc_sh.at[dring_v.at[slot + b]],
                                 ssem, add=True)
            for b in range(NB):
                i = g * NB + b
                pltpu.make_async_copy(rows_v.at[st, b],
                                      acc_sh.at[dring_v.at[slot + b]],
                                      ssem).wait()

                @pl.when(g + 2 < GROUPS)
                def _():
                    i2 = (g + 2) * NB + b
                    pltpu.async_copy(g_hbm.at[sidx_v.at[pl.ds(i2 * K, K)]],
                                     rows_v.at[st, b], gsems[st])

            @pl.when(g + RING < GROUPS)
            def _():
                pltpu.async_copy(dst3_hbm.at[wid, pl.ds((g + RING) * NB, NB)],
                                 dring_v.at[pl.ds(slot, NB)], isem)
        return carry

    lax.fori_loop(0, PAIRS, pair, 0)
    plsc.subcore_barrier()
    pltpu.sync_copy(acc_sh.at[pl.ds(r0, ROWS_PER_TILE)],
                    out_hbm.at[pl.ds(c * N_PAD + r0, ROWS_PER_TILE)])


_R = 1000


def _pre_body(degp_ref, x_ref, w1_ref, g_ref, dinv_ref):
    deg = degp_ref[0] + degp_ref[1] + 1.0
    dinv = lax.rsqrt(deg)
    dinv_ref[...] = dinv
    h = jnp.dot(x_ref[...], w1_ref[...], preferred_element_type=jnp.float32)
    g_ref[...] = h * dinv


def _mid_body(accp_ref, g1_ref, dinv_ref, b1_ref, w2_ref, g2_ref):
    t = accp_ref[0] + accp_ref[1] + g1_ref[...]
    dinv = dinv_ref[...]
    out1 = jnp.maximum(t * dinv + b1_ref[...], 0.0)
    h2 = jnp.dot(out1, w2_ref[...], preferred_element_type=jnp.float32)
    g2_ref[...] = h2 * dinv


def _post_body(accp_ref, g2_ref, dinv_ref, b2_ref, out_ref):
    t = accp_ref[0] + accp_ref[1] + g2_ref[...]
    out_ref[...] = (t * dinv_ref[...])[:, :64] + b2_ref[...]


def _pre_call(degp, x, w1):
    return pl.pallas_call(
        _pre_body,
        grid=(N // _R,),
        in_specs=[
            pl.BlockSpec((2, _R, 1), lambda i: (0, i, 0)),
            pl.BlockSpec((_R, 128), lambda i: (i, 0)),
            pl.BlockSpec((128, 128), lambda i: (0, 0)),
        ],
        out_specs=[
            pl.BlockSpec((_R, 128), lambda i: (i, 0)),
            pl.BlockSpec((_R, 1), lambda i: (i, 0)),
        ],
        out_shape=[
            jax.ShapeDtypeStruct((N, 128), jnp.float32),
            jax.ShapeDtypeStruct((N, 1), jnp.float32),
        ],
    )(degp, x, w1)


def _mid_call(accp, g1, dinv, b1, w2):
    return pl.pallas_call(
        _mid_body,
        grid=(N // _R,),
        in_specs=[
            pl.BlockSpec((2, _R, 128), lambda i: (0, i, 0)),
            pl.BlockSpec((_R, 128), lambda i: (i, 0)),
            pl.BlockSpec((_R, 1), lambda i: (i, 0)),
            pl.BlockSpec((1, 128), lambda i: (0, 0)),
            pl.BlockSpec((128, 128), lambda i: (0, 0)),
        ],
        out_specs=pl.BlockSpec((_R, 128), lambda i: (i, 0)),
        out_shape=jax.ShapeDtypeStruct((N, 128), jnp.float32),
    )(accp, g1, dinv, b1, w2)


def _post_call(accp, g2, dinv, b2):
    return pl.pallas_call(
        _post_body,
        grid=(N // _R,),
        in_specs=[
            pl.BlockSpec((2, _R, 128), lambda i: (0, i, 0)),
            pl.BlockSpec((_R, 128), lambda i: (i, 0)),
            pl.BlockSpec((_R, 1), lambda i: (i, 0)),
            pl.BlockSpec((1, 64), lambda i: (0, 0)),
        ],
        out_specs=pl.BlockSpec((_R, 64), lambda i: (i, 0)),
        out_shape=jax.ShapeDtypeStruct((N, 64), jnp.float32),
    )(accp, g2, dinv, b2)


def kernel(x, edge_index, W1, b1, W2, b2):
    npad = NW * EPAD
    spad = ((jnp.arange(npad, dtype=jnp.int32) * 97) % N).reshape(NW, -1)
    dpad = (N + (jnp.arange(npad, dtype=jnp.int32) % (N_PAD - N))).reshape(
        NW, -1)
    srcf = jnp.concatenate([edge_index[0].reshape(NW, EPW), spad], axis=1)
    dst3 = jnp.concatenate([edge_index[1].reshape(NW, EPW), dpad],
                           axis=1).reshape(NW, STEPS, K)
    dst3d = edge_index[1].reshape(NW, DRSTEPS, KD)
    npadd = NW * DPSTEPS * KD
    dpadd = (N + (jnp.arange(npadd, dtype=jnp.int32) % (N_PAD - N))).reshape(
        NW, DPSTEPS, KD)
    z128 = jnp.zeros((N_PAD, 128), jnp.float32)
    w2p = jnp.zeros((128, 128), jnp.float32).at[:, :64].set(W2)

    degp = _deg_kernel(dst3d, dpadd).reshape(NC, N_PAD, 1)
    g1, dinv = _pre_call(degp, x, W1)
    acc1 = _agg_kernel(g1, srcf, dst3, z128).reshape(NC, N_PAD, 128)
    g2 = _mid_call(acc1, g1, dinv, b1.reshape(1, 128), w2p)
    acc2 = _agg_kernel(g2, srcf, dst3, z128).reshape(NC, N_PAD, 128)
    return _post_call(acc2, g2, dinv, b2.reshape(1, 64))


# speedup vs baseline: 1.1039x; 1.1039x over previous
import functools

import jax
import jax.numpy as jnp
from jax import lax
from jax.experimental import pallas as pl
from jax.experimental.pallas import tpu as pltpu
from jax.experimental.pallas import tpu_sc as plsc

N = 10000
E = 320000
NC = 2
NS = 16
NW = NC * NS
N_PAD = 10240
ROWS_PER_TILE = N_PAD // NS
EPW = E // NW
KD = 80
DRSTEPS = EPW // KD
DSTEPS = 128
DPSTEPS = DSTEPS - DRSTEPS
K = 64
STEPS = 160
EPAD = STEPS * K - EPW
NB = 2
GROUPS = STEPS // NB
PAIRS = GROUPS // 2
RING = 4

_mesh = plsc.VectorSubcoreMesh(core_axis_name="c", subcore_axis_name="s")


@functools.partial(
    pl.kernel,
    out_type=jax.ShapeDtypeStruct((NC * N_PAD,), jnp.float32),
    mesh=_mesh,
    scratch_types=[
        pltpu.VMEM((DSTEPS, KD), jnp.int32),
        pltpu.VMEM((KD,), jnp.float32),
        pltpu.VMEM((ROWS_PER_TILE,), jnp.float32),
        pltpu.VMEM_SHARED((N_PAD,), jnp.float32),
        pltpu.SemaphoreType.DMA,
    ],
)
def _deg_kernel(dst_hbm, dpad_hbm, out_hbm, didx_v, ones_v, zero_v, hist_sh,
                sem):
    c = lax.axis_index("c")
    s = lax.axis_index("s")
    wid = s * NC + c
    for i in range(ROWS_PER_TILE // 16):
        zero_v[pl.ds(16 * i, 16)] = jnp.zeros((16,), jnp.float32)
    for i in range(KD // 16):
        ones_v[pl.ds(16 * i, 16)] = jnp.ones((16,), jnp.float32)
    pltpu.sync_copy(dst_hbm.at[wid], didx_v.at[pl.ds(0, DRSTEPS)])
    pltpu.sync_copy(dpad_hbm.at[wid], didx_v.at[pl.ds(DRSTEPS, DPSTEPS)])
    pltpu.sync_copy(zero_v, hist_sh.at[pl.ds(ROWS_PER_TILE * s, ROWS_PER_TILE)])
    plsc.subcore_barrier()

    def fire(i, carry):
        pltpu.async_copy(ones_v, hist_sh.at[didx_v.at[i]], sem, add=True)
        return carry

    lax.fori_loop(0, DSTEPS, fire, 0)

    def drain(i, carry):
        pltpu.make_async_copy(ones_v, hist_sh.at[didx_v.at[i]], sem).wait()
        return carry

    lax.fori_loop(0, DSTEPS, drain, 0)
    plsc.subcore_barrier()
    pltpu.sync_copy(
        hist_sh.at[pl.ds(ROWS_PER_TILE * s, ROWS_PER_TILE)],
        out_hbm.at[pl.ds(c * N_PAD + ROWS_PER_TILE * s, ROWS_PER_TILE)],
    )


@functools.partial(
    pl.kernel,
    out_type=jax.ShapeDtypeStruct((NC * N_PAD, 128), jnp.float32),
    mesh=_mesh,
    scratch_types=[
        pltpu.VMEM((STEPS * K,), jnp.int32),
        pltpu.VMEM((RING * NB, K), jnp.int32),
        pltpu.VMEM((2, NB, K, 128), jnp.float32),
        pltpu.VMEM_SHARED((N_PAD, 128), jnp.float32),
        pltpu.SemaphoreType.DMA,
        pltpu.SemaphoreType.DMA,
        pltpu.SemaphoreType.DMA,
        pltpu.SemaphoreType.DMA,
    ],
)
def _agg_kernel(g_hbm, srcf_hbm, dst3_hbm, z_hbm, out_hbm,
                sidx_v, dring_v, rows_v, acc_sh, gsem_a, gsem_b, ssem, isem):
    c = lax.axis_index("c")
    s = lax.axis_index("s")
    wid = s * NC + c
    r0 = ROWS_PER_TILE * s
    gsems = (gsem_a, gsem_b)

    pltpu.sync_copy(srcf_hbm.at[wid], sidx_v)
    pltpu.sync_copy(dst3_hbm.at[wid, pl.ds(0, RING * NB)], dring_v)

    @pl.when(c == 0)
    def _():
        @pl.when(s == NS - 1)
        def _():
            pltpu.sync_copy(g_hbm.at[pl.ds(r0, N - r0)],
                            acc_sh.at[pl.ds(r0, N - r0)])
            pltpu.sync_copy(z_hbm.at[pl.ds(0, N_PAD - N)],
                            acc_sh.at[pl.ds(N, N_PAD - N)])

        @pl.when(s < NS - 1)
        def _():
            pltpu.sync_copy(g_hbm.at[pl.ds(r0, ROWS_PER_TILE)],
                            acc_sh.at[pl.ds(r0, ROWS_PER_TILE)])

    @pl.when(c == 1)
    def _():
        pltpu.sync_copy(z_hbm.at[pl.ds(r0, ROWS_PER_TILE)],
                        acc_sh.at[pl.ds(r0, ROWS_PER_TILE)])
    plsc.subcore_barrier()

    for st in range(2):
        for b in range(NB):
            i = st * NB + b
            pltpu.async_copy(g_hbm.at[sidx_v.at[pl.ds(i * K, K)]],
                             rows_v.at[st, b], gsems[st])

    def pair(m, carry):
        for st in range(2):
            g = 2 * m + st
            slot = lax.rem(g, RING) * NB

            @pl.when(g >= RING)
            def _():
                pltpu.make_async_copy(
                    dst3_hbm.at[wid, pl.ds(g * NB, NB)],
                    dring_v.at[pl.ds(slot, NB)], isem).wait()
            for b in range(NB):
                i = g * NB + b
                pltpu.make_async_copy(g_hbm.at[sidx_v.at[pl.ds(i * K, K)]],
                                      rows_v.at[st, b], gsems[st]).wait()
                pltpu.async_copy(rows_v.at[st, b],
                                 acc_sh.at[dring_v.at[slot + b]],
                                 ssem, add=True)
            for b in range(NB):
                i = g * NB + b
                pltpu.make_async_copy(rows_v.at[st, b],
                                      acc_sh.at[dring_v.at[slot + b]],
                                      ssem).wait()

                @pl.when(g + 2 < GROUPS)
                def _():
                    i2 = (g + 2) * NB + b
                    pltpu.async_copy(g_hbm.at[sidx_v.at[pl.ds(i2 * K, K)]],
                                     rows_v.at[st, b], gsems[st])

            @pl.when(g + RING < GROUPS)
            def _():
                pltpu.async_copy(dst3_hbm.at[wid, pl.ds((g + RING) * NB, NB)],
                                 dring_v.at[pl.ds(slot, NB)], isem)
        return carry

    lax.fori_loop(0, PAIRS, pair, 0)
    plsc.subcore_barrier()
    pltpu.sync_copy(acc_sh.at[pl.ds(r0, ROWS_PER_TILE)],
                    out_hbm.at[pl.ds(c * N_PAD + r0, ROWS_PER_TILE)])


_R = 1000


def _pre_body(degp_ref, x_ref, w1_ref, g_ref, dinv_ref):
    deg = degp_ref[0] + degp_ref[1] + 1.0
    dinv = lax.rsqrt(deg)
    dinv_ref[...] = dinv
    h = jnp.dot(x_ref[...], w1_ref[...], preferred_element_type=jnp.float32)
    g_ref[...] = h * dinv


def _mid_body(accp_ref, dinv_ref, b1_ref, w2_ref, g2_ref):
    t = accp_ref[0] + accp_ref[1]
    dinv = dinv_ref[...]
    out1 = jnp.maximum(t * dinv + b1_ref[...], 0.0)
    h2 = jnp.dot(out1, w2_ref[...], preferred_element_type=jnp.float32)
    g2_ref[...] = h2 * dinv


def _post_body(accp_ref, dinv_ref, b2_ref, out_ref):
    t = accp_ref[0] + accp_ref[1]
    out_ref[...] = (t * dinv_ref[...])[:, :64] + b2_ref[...]


def _pre_call(degp, x, w1):
    return pl.pallas_call(
        _pre_body,
        grid=(N // _R,),
        in_specs=[
            pl.BlockSpec((2, _R, 1), lambda i: (0, i, 0)),
            pl.BlockSpec((_R, 128), lambda i: (i, 0)),
            pl.BlockSpec((128, 128), lambda i: (0, 0)),
        ],
        out_specs=[
            pl.BlockSpec((_R, 128), lambda i: (i, 0)),
            pl.BlockSpec((_R, 1), lambda i: (i, 0)),
        ],
        out_shape=[
            jax.ShapeDtypeStruct((N, 128), jnp.float32),
            jax.ShapeDtypeStruct((N, 1), jnp.float32),
        ],
    )(degp, x, w1)


def _mid_call(accp, dinv, b1, w2):
    return pl.pallas_call(
        _mid_body,
        grid=(N // _R,),
        in_specs=[
            pl.BlockSpec((2, _R, 128), lambda i: (0, i, 0)),
            pl.BlockSpec((_R, 1), lambda i: (i, 0)),
            pl.BlockSpec((1, 128), lambda i: (0, 0)),
            pl.BlockSpec((128, 128), lambda i: (0, 0)),
        ],
        out_specs=pl.BlockSpec((_R, 128), lambda i: (i, 0)),
        out_shape=jax.ShapeDtypeStruct((N, 128), jnp.float32),
    )(accp, dinv, b1, w2)


def _post_call(accp, dinv, b2):
    return pl.pallas_call(
        _post_body,
        grid=(N // _R,),
        in_specs=[
            pl.BlockSpec((2, _R, 128), lambda i: (0, i, 0)),
            pl.BlockSpec((_R, 1), lambda i: (i, 0)),
            pl.BlockSpec((1, 64), lambda i: (0, 0)),
        ],
        out_specs=pl.BlockSpec((_R, 64), lambda i: (i, 0)),
        out_shape=jax.ShapeDtypeStruct((N, 64), jnp.float32),
    )(accp, dinv, b2)


def kernel(x, edge_index, W1, b1, W2, b2):
    npad = NW * EPAD
    spad = ((jnp.arange(npad, dtype=jnp.int32) * 97) % N).reshape(NW, -1)
    dpad = (N + (jnp.arange(npad, dtype=jnp.int32) % (N_PAD - N))).reshape(
        NW, -1)
    srcf = jnp.concatenate([edge_index[0].reshape(NW, EPW), spad], axis=1)
    dst3 = jnp.concatenate([edge_index[1].reshape(NW, EPW), dpad],
                           axis=1).reshape(NW, STEPS, K)
    dst3d = edge_index[1].reshape(NW, DRSTEPS, KD)
    npadd = NW * DPSTEPS * KD
    dpadd = (N + (jnp.arange(npadd, dtype=jnp.int32) % (N_PAD - N))).reshape(
        NW, DPSTEPS, KD)
    z128 = jnp.zeros((N_PAD, 128), jnp.float32)
    w2p = jnp.zeros((128, 128), jnp.float32).at[:, :64].set(W2)

    degp = _deg_kernel(dst3d, dpadd).reshape(NC, N_PAD, 1)
    g1, dinv = _pre_call(degp, x, W1)
    acc1 = _agg_kernel(g1, srcf, dst3, z128).reshape(NC, N_PAD, 128)
    g2 = _mid_call(acc1, dinv, b1.reshape(1, 128), w2p)
    acc2 = _agg_kernel(g2, srcf, dst3, z128).reshape(NC, N_PAD, 128)
    return _post_call(acc2, dinv, b2.reshape(1, 64))


# speedup vs baseline: 1.1291x; 1.0228x over previous
import functools

import jax
import jax.numpy as jnp
import numpy as np
from jax import lax
from jax.experimental import pallas as pl
from jax.experimental.pallas import tpu as pltpu
from jax.experimental.pallas import tpu_sc as plsc

N = 10000
E = 320000
NC = 2
NS = 16
NW = NC * NS
N_PAD = 10240
ROWS_PER_TILE = N_PAD // NS
EPW = E // NW
KD = 80
DRSTEPS = EPW // KD
DSTEPS = 128
DPSTEPS = DSTEPS - DRSTEPS
K = 64
STEPS = 160
EPAD = STEPS * K - EPW
NB = 2
GROUPS = STEPS // NB
PAIRS = GROUPS // 2
RING = 4

_mesh = plsc.VectorSubcoreMesh(core_axis_name="c", subcore_axis_name="s")


@functools.partial(
    pl.kernel,
    out_type=jax.ShapeDtypeStruct((NC * N_PAD,), jnp.float32),
    mesh=_mesh,
    scratch_types=[
        pltpu.VMEM((DSTEPS, KD), jnp.int32),
        pltpu.VMEM((KD,), jnp.float32),
        pltpu.VMEM((ROWS_PER_TILE,), jnp.float32),
        pltpu.VMEM_SHARED((N_PAD,), jnp.float32),
        pltpu.SemaphoreType.DMA,
    ],
)
def _deg_kernel(dst_hbm, dpad_hbm, out_hbm, didx_v, ones_v, zero_v, hist_sh,
                sem):
    c = lax.axis_index("c")
    s = lax.axis_index("s")
    wid = s * NC + c
    for i in range(ROWS_PER_TILE // 16):
        zero_v[pl.ds(16 * i, 16)] = jnp.zeros((16,), jnp.float32)
    for i in range(KD // 16):
        ones_v[pl.ds(16 * i, 16)] = jnp.ones((16,), jnp.float32)
    pltpu.sync_copy(dst_hbm.at[wid], didx_v.at[pl.ds(0, DRSTEPS)])
    pltpu.sync_copy(dpad_hbm.at[wid], didx_v.at[pl.ds(DRSTEPS, DPSTEPS)])
    pltpu.sync_copy(zero_v, hist_sh.at[pl.ds(ROWS_PER_TILE * s, ROWS_PER_TILE)])
    plsc.subcore_barrier()

    def fire(i, carry):
        pltpu.async_copy(ones_v, hist_sh.at[didx_v.at[i]], sem, add=True)
        return carry

    lax.fori_loop(0, DSTEPS, fire, 0)

    def drain(i, carry):
        pltpu.make_async_copy(ones_v, hist_sh.at[didx_v.at[i]], sem).wait()
        return carry

    lax.fori_loop(0, DSTEPS, drain, 0)
    plsc.subcore_barrier()
    pltpu.sync_copy(
        hist_sh.at[pl.ds(ROWS_PER_TILE * s, ROWS_PER_TILE)],
        out_hbm.at[pl.ds(c * N_PAD + ROWS_PER_TILE * s, ROWS_PER_TILE)],
    )


@functools.partial(
    pl.kernel,
    out_type=jax.ShapeDtypeStruct((NC * N_PAD, 128), jnp.float32),
    mesh=_mesh,
    scratch_types=[
        pltpu.VMEM((STEPS * K,), jnp.int32),
        pltpu.VMEM((RING * NB, K), jnp.int32),
        pltpu.VMEM((2, NB, K, 128), jnp.float32),
        pltpu.VMEM_SHARED((N_PAD, 128), jnp.float32),
        pltpu.SemaphoreType.DMA,
        pltpu.SemaphoreType.DMA,
        pltpu.SemaphoreType.DMA,
        pltpu.SemaphoreType.DMA,
    ],
)
def _agg_kernel(g_hbm, srcf_hbm, dst3_hbm, z_hbm, out_hbm,
                sidx_v, dring_v, rows_v, acc_sh, gsem_a, gsem_b, ssem, isem):
    c = lax.axis_index("c")
    s = lax.axis_index("s")
    wid = s * NC + c
    r0 = ROWS_PER_TILE * s
    gsems = (gsem_a, gsem_b)

    pltpu.sync_copy(srcf_hbm.at[wid], sidx_v)
    pltpu.sync_copy(dst3_hbm.at[wid, pl.ds(0, RING * NB)], dring_v)

    @pl.when(c == 0)
    def _():
        @pl.when(s == NS - 1)
        def _():
            pltpu.sync_copy(g_hbm.at[pl.ds(r0, N - r0)],
                            acc_sh.at[pl.ds(r0, N - r0)])
            pltpu.sync_copy(z_hbm.at[pl.ds(0, N_PAD - N)],
                            acc_sh.at[pl.ds(N, N_PAD - N)])

        @pl.when(s < NS - 1)
        def _():
            pltpu.sync_copy(g_hbm.at[pl.ds(r0, ROWS_PER_TILE)],
                            acc_sh.at[pl.ds(r0, ROWS_PER_TILE)])

    @pl.when(c == 1)
    def _():
        pltpu.sync_copy(z_hbm.at[pl.ds(r0, ROWS_PER_TILE)],
                        acc_sh.at[pl.ds(r0, ROWS_PER_TILE)])
    plsc.subcore_barrier()

    for st in range(2):
        for b in range(NB):
            i = st * NB + b
            pltpu.async_copy(g_hbm.at[sidx_v.at[pl.ds(i * K, K)]],
                             rows_v.at[st, b], gsems[st])

    def pair(m, carry):
        for st in range(2):
            g = 2 * m + st
            slot = lax.rem(g, RING) * NB

            @pl.when(g >= RING)
            def _():
                pltpu.make_async_copy(
                    dst3_hbm.at[wid, pl.ds(g * NB, NB)],
                    dring_v.at[pl.ds(slot, NB)], isem).wait()
            for b in range(NB):
                i = g * NB + b
                pltpu.make_async_copy(g_hbm.at[sidx_v.at[pl.ds(i * K, K)]],
                                      rows_v.at[st, b], gsems[st]).wait()
                pltpu.async_copy(rows_v.at[st, b],
                                 acc_sh.at[dring_v.at[slot + b]],
                                 ssem, add=True)
            for b in range(NB):
                i = g * NB + b
                pltpu.make_async_copy(rows_v.at[st, b],
                                      acc_sh.at[dring_v.at[slot + b]],
                                      ssem).wait()

                @pl.when(g + 2 < GROUPS)
                def _():
                    i2 = (g + 2) * NB + b
                    pltpu.async_copy(g_hbm.at[sidx_v.at[pl.ds(i2 * K, K)]],
                                     rows_v.at[st, b], gsems[st])

            @pl.when(g + RING < GROUPS)
            def _():
                pltpu.async_copy(dst3_hbm.at[wid, pl.ds((g + RING) * NB, NB)],
                                 dring_v.at[pl.ds(slot, NB)], isem)
        return carry

    lax.fori_loop(0, PAIRS, pair, 0)
    plsc.subcore_barrier()
    pltpu.sync_copy(acc_sh.at[pl.ds(r0, ROWS_PER_TILE)],
                    out_hbm.at[pl.ds(c * N_PAD + r0, ROWS_PER_TILE)])


_R = 2000


def _pre_body(degp_ref, x_ref, w1_ref, g_ref, dinv_ref):
    deg = degp_ref[0] + degp_ref[1] + 1.0
    dinv = lax.rsqrt(deg)
    dinv_ref[...] = dinv
    h = jnp.dot(x_ref[...], w1_ref[...], preferred_element_type=jnp.float32)
    g_ref[...] = h * dinv


def _mid_body(accp_ref, dinv_ref, b1_ref, w2_ref, g2_ref):
    t = accp_ref[0] + accp_ref[1]
    dinv = dinv_ref[...]
    out1 = jnp.maximum(t * dinv + b1_ref[...], 0.0)
    h2 = jnp.dot(out1, w2_ref[...], preferred_element_type=jnp.float32)
    g2_ref[...] = h2 * dinv


def _post_body(accp_ref, dinv_ref, b2_ref, out_ref):
    t = accp_ref[0] + accp_ref[1]
    out_ref[...] = (t * dinv_ref[...])[:, :64] + b2_ref[...]


def _pre_call(degp, x, w1):
    return pl.pallas_call(
        _pre_body,
        grid=(N // _R,),
        in_specs=[
            pl.BlockSpec((2, _R, 1), lambda i: (0, i, 0)),
            pl.BlockSpec((_R, 128), lambda i: (i, 0)),
            pl.BlockSpec((128, 128), lambda i: (0, 0)),
        ],
        out_specs=[
            pl.BlockSpec((_R, 128), lambda i: (i, 0)),
            pl.BlockSpec((_R, 1), lambda i: (i, 0)),
        ],
        out_shape=[
            jax.ShapeDtypeStruct((N, 128), jnp.float32),
            jax.ShapeDtypeStruct((N, 1), jnp.float32),
        ],
    )(degp, x, w1)


def _mid_call(accp, dinv, b1, w2):
    return pl.pallas_call(
        _mid_body,
        grid=(N // _R,),
        in_specs=[
            pl.BlockSpec((2, _R, 128), lambda i: (0, i, 0)),
            pl.BlockSpec((_R, 1), lambda i: (i, 0)),
            pl.BlockSpec((1, 128), lambda i: (0, 0)),
            pl.BlockSpec((128, 128), lambda i: (0, 0)),
        ],
        out_specs=pl.BlockSpec((_R, 128), lambda i: (i, 0)),
        out_shape=jax.ShapeDtypeStruct((N, 128), jnp.float32),
    )(accp, dinv, b1, w2)


def _post_call(accp, dinv, b2):
    return pl.pallas_call(
        _post_body,
        grid=(N // _R,),
        in_specs=[
            pl.BlockSpec((2, _R, 128), lambda i: (0, i, 0)),
            pl.BlockSpec((_R, 1), lambda i: (i, 0)),
            pl.BlockSpec((1, 64), lambda i: (0, 0)),
        ],
        out_specs=pl.BlockSpec((_R, 64), lambda i: (i, 0)),
        out_shape=jax.ShapeDtypeStruct((N, 64), jnp.float32),
    )(accp, dinv, b2)


def kernel(x, edge_index, W1, b1, W2, b2):
    npad = NW * EPAD
    spad = jnp.asarray(((np.arange(npad, dtype=np.int32) * 97) % N)
                       .reshape(NW, -1))
    dpad = jnp.asarray((N + (np.arange(npad, dtype=np.int32) % (N_PAD - N)))
                       .reshape(NW, -1))
    srcf = jnp.concatenate([edge_index[0].reshape(NW, EPW), spad], axis=1)
    dst3 = jnp.concatenate([edge_index[1].reshape(NW, EPW), dpad],
                           axis=1).reshape(NW, STEPS, K)
    dst3d = edge_index[1].reshape(NW, DRSTEPS, KD)
    npadd = NW * DPSTEPS * KD
    dpadd = jnp.asarray(
        (N + (np.arange(npadd, dtype=np.int32) % (N_PAD - N)))
        .reshape(NW, DPSTEPS, KD))
    z128 = jnp.asarray(np.zeros((N_PAD, 128), np.float32))
    w2p = jnp.zeros((128, 128), jnp.float32).at[:, :64].set(W2)

    degp = _deg_kernel(dst3d, dpadd).reshape(NC, N_PAD, 1)
    g1, dinv = _pre_call(degp, x, W1)
    acc1 = _agg_kernel(g1, srcf, dst3, z128).reshape(NC, N_PAD, 128)
    g2 = _mid_call(acc1, dinv, b1.reshape(1, 128), w2p)
    acc2 = _agg_kernel(g2, srcf, dst3, z128).reshape(NC, N_PAD, 128)
    return _post_call(acc2, dinv, b2.reshape(1, 64))
